# baseline (speedup 1.0000x reference)
import functools

import jax
import jax.numpy as jnp
from jax.experimental import pallas as pl
from jax.experimental.pallas import tpu as pltpu


def _fused_kernel(x_hbm, w1_ref, b1_ref, w2_ref, b2_ref, v_ref, r_ref,
                  c_ref, out_hbm, xbuf, ybuf, in_sem, out_sem, *,
                  d2, nnz, chunk, n_chunks):
    def _mk_in(k):
        return (x_hbm.at[pl.ds(k * chunk, chunk), :], xbuf.at[k],
                in_sem.at[k])

    def in_start(k):
        if k % 4 == 0:
            pltpu.make_async_copy(*_mk_in(k)).start()
        elif k % 4 == 1:
            pltpu.make_async_copy(*_mk_in(k)).start()
        elif k % 4 == 2:
            pltpu.make_async_copy(*_mk_in(k)).start()
        else:
            pltpu.make_async_copy(*_mk_in(k)).start()

    def in_wait(k):
        if k % 4 == 0:
            pltpu.make_async_copy(*_mk_in(k)).wait()
        elif k % 4 == 1:
            pltpu.make_async_copy(*_mk_in(k)).wait()
        elif k % 4 == 2:
            pltpu.make_async_copy(*_mk_in(k)).wait()
        else:
            pltpu.make_async_copy(*_mk_in(k)).wait()

    def out_copy(k):
        return pltpu.make_async_copy(
            ybuf.at[k],
            out_hbm.at[:, pl.ds(k * chunk, chunk)],
            out_sem.at[k])

    for k in range(n_chunks):
        in_start(k)

    iota = jax.lax.broadcasted_iota(jnp.int32, (d2, nnz), 0)
    rt = (r_ref[0:1, :] == iota).astype(jnp.float32)
    ct = (c_ref[0:1, :] == iota).astype(jnp.float32)
    st = jax.lax.dot_general(
        ct * v_ref[0:1, :], rt,
        (((1,), (1,)), ((), ())),
        preferred_element_type=jnp.float32)
    w12 = jnp.dot(w1_ref[...], w2_ref[...],
                  preferred_element_type=jnp.float32)
    wf = jnp.dot(w12, st, preferred_element_type=jnp.float32)
    bvec = jnp.dot(b1_ref[...], w2_ref[...],
                   preferred_element_type=jnp.float32) + b2_ref[...]
    bf_col = jax.lax.dot_general(
        st, bvec,
        (((0,), (1,)), ((), ())),
        preferred_element_type=jnp.float32)

    for k in range(n_chunks):
        in_wait(k)
        y_t = jax.lax.dot_general(
            wf, xbuf[k],
            (((0,), (1,)), ((), ())),
            preferred_element_type=jnp.float32)
        ybuf[k] = y_t + bf_col
        out_copy(k).start()

    for k in range(n_chunks):
        out_copy(k).wait()


@jax.jit
def kernel(inputs, W1, b1, W2, b2, sp_vals, sp_rows, sp_cols):
    batch, d_in = inputs.shape
    d1 = W1.shape[1]
    d2 = W2.shape[1]
    nnz = sp_vals.shape[0]

    chunk = 1024
    n_chunks = batch // chunk

    full = lambda shape: pl.BlockSpec(shape, lambda: (0, 0))
    out = pl.pallas_call(
        functools.partial(_fused_kernel, d2=d2, nnz=nnz, chunk=chunk,
                          n_chunks=n_chunks),
        in_specs=[
            pl.BlockSpec(memory_space=pltpu.MemorySpace.HBM),
            full((d_in, d1)),
            full((1, d1)),
            full((d1, d2)),
            full((1, d2)),
            full((1, nnz)),
            full((1, nnz)),
            full((1, nnz)),
        ],
        out_specs=pl.BlockSpec(memory_space=pltpu.MemorySpace.HBM),
        out_shape=jax.ShapeDtypeStruct((d2, batch), jnp.float32),
        scratch_shapes=[
            pltpu.VMEM((n_chunks, chunk, d_in), jnp.float32),
            pltpu.VMEM((n_chunks, d2, chunk), jnp.float32),
            pltpu.SemaphoreType.DMA((n_chunks,)),
            pltpu.SemaphoreType.DMA((n_chunks,)),
        ],
    )(inputs, W1, b1.reshape(1, d1), W2, b2.reshape(1, d2),
      sp_vals.reshape(1, nnz), sp_rows.reshape(1, nnz),
      sp_cols.reshape(1, nnz))
    return out.T


# speedup vs baseline: 1.0680x; 1.0680x over previous
import functools

import jax
import jax.numpy as jnp
from jax.experimental import pallas as pl
from jax.experimental.pallas import tpu as pltpu


def _fused_kernel(x_hbm, w1_ref, b1_ref, w2_ref, b2_ref, v_ref, r_ref,
                  c_ref, out_hbm, xbuf, ybuf, in_sem, out_sem, *,
                  d2, nnz, chunk, n_chunks):
    def _mk_in(k):
        return (x_hbm.at[pl.ds(k * chunk, chunk), :], xbuf.at[k],
                in_sem.at[k])

    def in_start(k):
        if k % 4 == 0:
            pltpu.make_async_copy(*_mk_in(k)).start()
        elif k % 4 == 1:
            pltpu.make_async_copy(*_mk_in(k)).start()
        elif k % 4 == 2:
            pltpu.make_async_copy(*_mk_in(k)).start()
        else:
            pltpu.make_async_copy(*_mk_in(k)).start()

    def in_wait(k):
        if k % 4 == 0:
            pltpu.make_async_copy(*_mk_in(k)).wait()
        elif k % 4 == 1:
            pltpu.make_async_copy(*_mk_in(k)).wait()
        elif k % 4 == 2:
            pltpu.make_async_copy(*_mk_in(k)).wait()
        else:
            pltpu.make_async_copy(*_mk_in(k)).wait()

    def out_copy(k):
        return pltpu.make_async_copy(
            ybuf.at[k],
            out_hbm.at[:, pl.ds(k * chunk, chunk)],
            out_sem.at[k])

    for k in range(n_chunks):
        in_start(k)

    iota = jax.lax.broadcasted_iota(jnp.int32, (d2, nnz), 0)
    rt = (r_ref[0:1, :] == iota).astype(jnp.float32)
    ct = (c_ref[0:1, :] == iota).astype(jnp.float32)
    st = jax.lax.dot_general(
        ct * v_ref[0:1, :], rt,
        (((1,), (1,)), ((), ())),
        preferred_element_type=jnp.float32)
    w12 = jnp.dot(w1_ref[...], w2_ref[...],
                  preferred_element_type=jnp.float32)
    wf = jnp.dot(w12, st, preferred_element_type=jnp.float32)
    bvec = jnp.dot(b1_ref[...], w2_ref[...],
                   preferred_element_type=jnp.float32) + b2_ref[...]
    bf_col = jax.lax.dot_general(
        st, bvec,
        (((0,), (1,)), ((), ())),
        preferred_element_type=jnp.float32)

    for k in range(n_chunks):
        in_wait(k)
        y_t = jax.lax.dot_general(
            wf, xbuf[k],
            (((0,), (1,)), ((), ())),
            preferred_element_type=jnp.float32)
        ybuf[k] = y_t + bf_col
        out_copy(k).start()

    for k in range(n_chunks):
        out_copy(k).wait()


@jax.jit
def kernel(inputs, W1, b1, W2, b2, sp_vals, sp_rows, sp_cols):
    batch, d_in = inputs.shape
    d1 = W1.shape[1]
    d2 = W2.shape[1]
    nnz = sp_vals.shape[0]

    chunk = 2048
    n_chunks = batch // chunk

    full = lambda shape: pl.BlockSpec(shape, lambda: (0, 0))
    out = pl.pallas_call(
        functools.partial(_fused_kernel, d2=d2, nnz=nnz, chunk=chunk,
                          n_chunks=n_chunks),
        in_specs=[
            pl.BlockSpec(memory_space=pltpu.MemorySpace.HBM),
            full((d_in, d1)),
            full((1, d1)),
            full((d1, d2)),
            full((1, d2)),
            full((1, nnz)),
            full((1, nnz)),
            full((1, nnz)),
        ],
        out_specs=pl.BlockSpec(memory_space=pltpu.MemorySpace.HBM),
        out_shape=jax.ShapeDtypeStruct((d2, batch), jnp.float32),
        scratch_shapes=[
            pltpu.VMEM((n_chunks, chunk, d_in), jnp.float32),
            pltpu.VMEM((n_chunks, d2, chunk), jnp.float32),
            pltpu.SemaphoreType.DMA((n_chunks,)),
            pltpu.SemaphoreType.DMA((n_chunks,)),
        ],
    )(inputs, W1, b1.reshape(1, d1), W2, b2.reshape(1, d2),
      sp_vals.reshape(1, nnz), sp_rows.reshape(1, nnz),
      sp_cols.reshape(1, nnz))
    return out.T
